# Initial kernel scaffold; baseline (speedup 1.0000x reference)
import jax
import jax.numpy as jnp
from jax.experimental import pallas as pl


def kernel(x, edge_index, W1, b1, W2, b2, W3, b3):
    raise NotImplementedError("write your pallas kernel here")


# speedup vs baseline: 7.3768x; 7.3768x over previous
import functools

import jax
import jax.numpy as jnp
from jax import lax
from jax.experimental import pallas as pl
from jax.experimental.pallas import tpu as pltpu
from jax.experimental.pallas import tpu_sc as plsc

N = 10000
E = 320000
D = 128

NPAD = 10240
CB = 128
NCHUNK = 80
EPT = NCHUNK * CB
EPAD = 32 * EPT
ROWS_PER_TILE = NPAD // 16

_mesh = plsc.VectorSubcoreMesh(core_axis_name="c", subcore_axis_name="s")


@functools.partial(
    pl.kernel,
    mesh=_mesh,
    out_type=jax.ShapeDtypeStruct((2, NPAD, 16), jnp.float32),
    scratch_types=[
        pltpu.VMEM((NCHUNK, CB), jnp.int32),
        pltpu.VMEM((CB, 16), jnp.float32),
        pltpu.VMEM((64, 16), jnp.float32),
        pltpu.VMEM_SHARED((NPAD, 16), jnp.float32),
    ],
)
def _deg_kernel(col_hbm, deg_out, col_v, ones_v, zero_v, deg_sh):
    c = lax.axis_index("c")
    s = lax.axis_index("s")
    wid = s * 2 + c
    for i in range(64):
        zero_v[i, pl.ds(0, 16)] = jnp.zeros((16,), jnp.float32)
    for i in range(CB):
        ones_v[i, pl.ds(0, 16)] = jnp.ones((16,), jnp.float32)

    def zero_body(j, carry):
        pltpu.sync_copy(zero_v, deg_sh.at[pl.ds(s * ROWS_PER_TILE + j * 64, 64)])
        return carry

    lax.fori_loop(0, ROWS_PER_TILE // 64, zero_body, 0)
    pltpu.sync_copy(col_hbm.at[pl.ds(wid * NCHUNK, NCHUNK)], col_v)
    plsc.subcore_barrier()

    def hist_body(j, carry):
        pltpu.sync_copy(ones_v, deg_sh.at[col_v.at[j]], add=True)
        return carry

    lax.fori_loop(0, NCHUNK, hist_body, 0)
    plsc.subcore_barrier()
    pltpu.sync_copy(deg_sh.at[pl.ds(s * ROWS_PER_TILE, ROWS_PER_TILE)],
                    deg_out.at[c, pl.ds(s * ROWS_PER_TILE, ROWS_PER_TILE)])


@functools.partial(
    pl.kernel,
    mesh=_mesh,
    out_type=jax.ShapeDtypeStruct((2, NPAD, D), jnp.float32),
    scratch_types=[
        pltpu.VMEM((NCHUNK, CB), jnp.int32),
        pltpu.VMEM((NCHUNK, CB), jnp.int32),
        pltpu.VMEM((CB, D), jnp.float32),
        pltpu.VMEM((16, D), jnp.float32),
        pltpu.VMEM_SHARED((NPAD, D), jnp.float32),
        pltpu.SemaphoreType.DMA,
    ],
)
def _edge_kernel(row_hbm, col_hbm, t_hbm, out_hbm,
                 row_v, col_v, rows_v, zero_v, acc_sh, sem):
    c = lax.axis_index("c")
    s = lax.axis_index("s")
    wid = s * 2 + c

    for i in range(16):
        for d in range(D // 16):
            zero_v[i, pl.ds(d * 16, 16)] = jnp.zeros((16,), jnp.float32)

    def zero_body(j, carry):
        pltpu.sync_copy(zero_v, acc_sh.at[pl.ds(s * ROWS_PER_TILE + j * 16, 16)])
        return carry

    lax.fori_loop(0, ROWS_PER_TILE // 16, zero_body, 0)

    pltpu.sync_copy(row_hbm.at[pl.ds(wid * NCHUNK, NCHUNK)], row_v)
    pltpu.sync_copy(col_hbm.at[pl.ds(wid * NCHUNK, NCHUNK)], col_v)
    plsc.subcore_barrier()

    def body(j, carry):
        pltpu.async_copy(t_hbm.at[row_v.at[j]], rows_v, sem).wait()
        pltpu.sync_copy(rows_v, acc_sh.at[col_v.at[j]], add=True)
        return carry

    lax.fori_loop(0, NCHUNK, body, 0)
    plsc.subcore_barrier()
    pltpu.sync_copy(acc_sh.at[pl.ds(s * ROWS_PER_TILE, ROWS_PER_TILE)],
                    out_hbm.at[c, pl.ds(s * ROWS_PER_TILE, ROWS_PER_TILE)])


def _tc_first_body(x_ref, w_ref, degp_ref, t_ref, dis_ref):
    deg = degp_ref[0, :, 0] + degp_ref[1, :, 0] + 1.0
    dis = lax.rsqrt(deg)
    t = jnp.dot(x_ref[...], w_ref[...], preferred_element_type=jnp.float32,
                precision=lax.Precision.HIGHEST)
    t_ref[...] = t * dis[:, None]
    dis_ref[...] = dis


def _tc_mid_body(p_ref, tp_ref, dis_ref, b_ref, w_ref, out_ref):
    dis = dis_ref[...]
    h = dis[:, None] * (p_ref[0] + p_ref[1] + tp_ref[...]) + b_ref[...]
    t = jnp.dot(h, w_ref[...], preferred_element_type=jnp.float32,
                precision=lax.Precision.HIGHEST)
    out_ref[...] = t * dis[:, None]


def _tc_final_body(p_ref, tp_ref, dis_ref, b_ref, out_ref):
    dis = dis_ref[...]
    out_ref[...] = dis[:, None] * (p_ref[0] + p_ref[1] + tp_ref[...]) + b_ref[...]


_BN = 1024
_GRID = NPAD // _BN

_tc_first = pl.pallas_call(
    _tc_first_body,
    grid=(_GRID,),
    in_specs=[
        pl.BlockSpec((_BN, D), lambda i: (i, 0)),
        pl.BlockSpec((D, D), lambda i: (0, 0)),
        pl.BlockSpec((2, _BN, 16), lambda i: (0, i, 0)),
    ],
    out_specs=[
        pl.BlockSpec((_BN, D), lambda i: (i, 0)),
        pl.BlockSpec((_BN,), lambda i: (i,)),
    ],
    out_shape=[
        jax.ShapeDtypeStruct((NPAD, D), jnp.float32),
        jax.ShapeDtypeStruct((NPAD,), jnp.float32),
    ],
)

_tc_mid = pl.pallas_call(
    _tc_mid_body,
    grid=(_GRID,),
    in_specs=[
        pl.BlockSpec((2, _BN, D), lambda i: (0, i, 0)),
        pl.BlockSpec((_BN, D), lambda i: (i, 0)),
        pl.BlockSpec((_BN,), lambda i: (i,)),
        pl.BlockSpec((1, D), lambda i: (0, 0)),
        pl.BlockSpec((D, D), lambda i: (0, 0)),
    ],
    out_specs=pl.BlockSpec((_BN, D), lambda i: (i, 0)),
    out_shape=jax.ShapeDtypeStruct((NPAD, D), jnp.float32),
)

_tc_final = pl.pallas_call(
    _tc_final_body,
    grid=(_GRID,),
    in_specs=[
        pl.BlockSpec((2, _BN, D), lambda i: (0, i, 0)),
        pl.BlockSpec((_BN, D), lambda i: (i, 0)),
        pl.BlockSpec((_BN,), lambda i: (i,)),
        pl.BlockSpec((1, D), lambda i: (0, 0)),
    ],
    out_specs=pl.BlockSpec((_BN, D), lambda i: (i, 0)),
    out_shape=jax.ShapeDtypeStruct((NPAD, D), jnp.float32),
)


def kernel(x, edge_index, W1, b1, W2, b2, W3, b3):
    row = edge_index[0]
    col = edge_index[1]
    pad = jnp.full((EPAD - E,), N, jnp.int32)
    rowp = jnp.concatenate([row, pad])
    colp = jnp.concatenate([col, pad])
    row2d = rowp.reshape(32 * NCHUNK, CB)
    col2d = colp.reshape(32 * NCHUNK, CB)
    xpad = jnp.concatenate([x, jnp.zeros((NPAD - N, D), jnp.float32)])

    degp = _deg_kernel(col2d)
    t1, dis = _tc_first(xpad, W1, degp)
    s1 = _edge_kernel(row2d, col2d, t1)
    t2 = _tc_mid(s1, t1, dis, b1.reshape(1, D), W2)
    s2 = _edge_kernel(row2d, col2d, t2)
    t3 = _tc_mid(s2, t2, dis, b2.reshape(1, D), W3)
    s3 = _edge_kernel(row2d, col2d, t3)
    out = _tc_final(s3, t3, dis, b3.reshape(1, D))
    return out[:N]
